# Initial kernel scaffold; baseline (speedup 1.0000x reference)
import jax
import jax.numpy as jnp
from jax.experimental import pallas as pl


def kernel(peptide_tokens, decoration, charge, a_emb, charge_emb, phos_emb, pos_emb, ln_gamma, ln_beta):
    raise NotImplementedError("write your pallas kernel here")


# speedup vs baseline: 2.7499x; 2.7499x over previous
import functools

import jax
import jax.numpy as jnp
from jax.experimental import pallas as pl
from jax.experimental.pallas import tpu as pltpu

N, L, H = 1024, 50, 768
LN_EPS = 1e-12
BT = 800
K = 64


def _body(tok_ref, dec_ref, chg_ref, table_ref, pos_ref, gamma_ref, beta_ref,
          out_ref):
    tok = tok_ref[...]
    dec = dec_ref[...]
    chg = chg_ref[...]
    dec = dec - 5 * (dec == 5).astype(dec.dtype)
    col = jax.lax.broadcasted_iota(jnp.int32, (BT, K), 1)
    hot = (col == tok) | (col == dec + 30) | (col == chg + 40)
    hot = hot.astype(jnp.float32)
    emb = jnp.dot(hot, table_ref[...], preferred_element_type=jnp.float32)
    emb = emb + pos_ref[...]
    mean = jnp.mean(emb, axis=-1, keepdims=True)
    cent = emb - mean
    var = jnp.mean(cent * cent, axis=-1, keepdims=True)
    y = cent * jax.lax.rsqrt(var + LN_EPS)
    out_ref[...] = y * gamma_ref[...] + beta_ref[...]


@jax.jit
def kernel(peptide_tokens, decoration, charge, a_emb, charge_emb, phos_emb,
           pos_emb, ln_gamma, ln_beta):
    n, l = peptide_tokens.shape
    h = a_emb.shape[1]
    nt = n * l
    tok_c = peptide_tokens.reshape(nt, 1).astype(jnp.int32)
    dec_c = decoration.reshape(nt, 1).astype(jnp.int32)
    chg_c = jnp.broadcast_to(charge[:, None], (n, l)).reshape(nt, 1)
    chg_c = chg_c.astype(jnp.int32)
    table = jnp.concatenate(
        [a_emb, phos_emb, charge_emb,
         jnp.zeros((K - 50, h), jnp.float32)], axis=0)
    pos_tile = jnp.tile(pos_emb[:l], (BT // l, 1))
    gamma = ln_gamma.reshape(1, h)
    beta = ln_beta.reshape(1, h)

    grid = (nt // BT,)
    out = pl.pallas_call(
        _body,
        grid=grid,
        in_specs=[
            pl.BlockSpec((BT, 1), lambda i: (i, 0)),
            pl.BlockSpec((BT, 1), lambda i: (i, 0)),
            pl.BlockSpec((BT, 1), lambda i: (i, 0)),
            pl.BlockSpec((K, h), lambda i: (0, 0)),
            pl.BlockSpec((BT, h), lambda i: (0, 0)),
            pl.BlockSpec((1, h), lambda i: (0, 0)),
            pl.BlockSpec((1, h), lambda i: (0, 0)),
        ],
        out_specs=pl.BlockSpec((BT, h), lambda i: (i, 0)),
        out_shape=jax.ShapeDtypeStruct((nt, h), jnp.float32),
        compiler_params=pltpu.CompilerParams(
            dimension_semantics=("arbitrary",),
        ),
    )(tok_c, dec_c, chg_c, table, pos_tile, gamma, beta)
    return out.reshape(n, l, h)


# speedup vs baseline: 4.2303x; 1.5383x over previous
import functools

import jax
import jax.numpy as jnp
from jax.experimental import pallas as pl
from jax.experimental.pallas import tpu as pltpu

N, L, H = 1024, 50, 768
LN_EPS = 1e-12
BT = 800
K = 64


def _body(tok_ref, dec_ref, chg_ref, table_ref, pos_ref, gamma_ref, beta_ref,
          out_ref):
    tok = tok_ref[...]
    dec = dec_ref[...]
    chg = chg_ref[...]
    dec = dec - 5 * (dec == 5).astype(dec.dtype)
    col = jax.lax.broadcasted_iota(jnp.int32, (BT, K), 1)
    hot = (col == tok) | (col == dec + 30) | (col == chg + 40)
    hot = hot.astype(jnp.float32)
    emb = jnp.dot(hot, table_ref[...], preferred_element_type=jnp.float32)
    emb = emb + pos_ref[...]
    mean = jnp.mean(emb, axis=-1, keepdims=True)
    cent = emb - mean
    var = jnp.mean(cent * cent, axis=-1, keepdims=True)
    y = cent * jax.lax.rsqrt(var + LN_EPS)
    y = y * gamma_ref[...] + beta_ref[...]
    out_ref[...] = y.reshape(BT // L, L, H)


@jax.jit
def kernel(peptide_tokens, decoration, charge, a_emb, charge_emb, phos_emb,
           pos_emb, ln_gamma, ln_beta):
    n, l = peptide_tokens.shape
    h = a_emb.shape[1]
    nt = n * l
    tok_c = peptide_tokens.reshape(nt, 1).astype(jnp.int32)
    dec_c = decoration.reshape(nt, 1).astype(jnp.int32)
    chg_c = jnp.broadcast_to(charge[:, None], (n, l)).reshape(nt, 1)
    chg_c = chg_c.astype(jnp.int32)
    table = jnp.concatenate(
        [a_emb, phos_emb, charge_emb,
         jnp.zeros((K - 50, h), jnp.float32)], axis=0)
    pos_tile = jnp.tile(pos_emb[:l], (BT // l, 1))
    gamma = ln_gamma.reshape(1, h)
    beta = ln_beta.reshape(1, h)

    bn = BT // l
    grid = (nt // BT,)
    out = pl.pallas_call(
        _body,
        grid=grid,
        in_specs=[
            pl.BlockSpec((BT, 1), lambda i: (i, 0)),
            pl.BlockSpec((BT, 1), lambda i: (i, 0)),
            pl.BlockSpec((BT, 1), lambda i: (i, 0)),
            pl.BlockSpec((K, h), lambda i: (0, 0)),
            pl.BlockSpec((BT, h), lambda i: (0, 0)),
            pl.BlockSpec((1, h), lambda i: (0, 0)),
            pl.BlockSpec((1, h), lambda i: (0, 0)),
        ],
        out_specs=pl.BlockSpec((bn, l, h), lambda i: (i, 0, 0)),
        out_shape=jax.ShapeDtypeStruct((n, l, h), jnp.float32),
        compiler_params=pltpu.CompilerParams(
            dimension_semantics=("arbitrary",),
        ),
    )(tok_c, dec_c, chg_c, table, pos_tile, gamma, beta)
    return out


# speedup vs baseline: 4.3832x; 1.0361x over previous
import functools

import jax
import jax.numpy as jnp
from jax.experimental import pallas as pl
from jax.experimental.pallas import tpu as pltpu

N, L, H = 1024, 50, 768
LN_EPS = 1e-12
BT = 1600
K = 64


def _body(tok_ref, dec_ref, chg_ref, table_ref, pos_ref, gamma_ref, beta_ref,
          out_ref):
    tok = tok_ref[...]
    dec = dec_ref[...]
    chg = chg_ref[...]
    dec = dec - 5 * (dec == 5).astype(dec.dtype)
    col = jax.lax.broadcasted_iota(jnp.int32, (BT, K), 1)
    hot = (col == tok) | (col == dec + 30) | (col == chg + 40)
    hot = hot.astype(jnp.float32)
    emb = jnp.dot(hot, table_ref[...], preferred_element_type=jnp.float32)
    emb = emb + pos_ref[...]
    mean = jnp.mean(emb, axis=-1, keepdims=True)
    cent = emb - mean
    var = jnp.mean(cent * cent, axis=-1, keepdims=True)
    y = cent * jax.lax.rsqrt(var + LN_EPS)
    y = y * gamma_ref[...] + beta_ref[...]
    out_ref[...] = y.reshape(BT // L, L, H)


@jax.jit
def kernel(peptide_tokens, decoration, charge, a_emb, charge_emb, phos_emb,
           pos_emb, ln_gamma, ln_beta):
    n, l = peptide_tokens.shape
    h = a_emb.shape[1]
    nt = n * l
    tok_c = peptide_tokens.reshape(nt, 1).astype(jnp.int32)
    dec_c = decoration.reshape(nt, 1).astype(jnp.int32)
    chg_c = jnp.broadcast_to(charge[:, None], (n, l)).reshape(nt, 1)
    chg_c = chg_c.astype(jnp.int32)
    table = jnp.concatenate(
        [a_emb, phos_emb, charge_emb,
         jnp.zeros((K - 50, h), jnp.float32)], axis=0)
    pos_tile = jnp.tile(pos_emb[:l], (BT // l, 1))
    gamma = ln_gamma.reshape(1, h)
    beta = ln_beta.reshape(1, h)

    bn = BT // l
    grid = (nt // BT,)
    out = pl.pallas_call(
        _body,
        grid=grid,
        in_specs=[
            pl.BlockSpec((BT, 1), lambda i: (i, 0)),
            pl.BlockSpec((BT, 1), lambda i: (i, 0)),
            pl.BlockSpec((BT, 1), lambda i: (i, 0)),
            pl.BlockSpec((K, h), lambda i: (0, 0)),
            pl.BlockSpec((BT, h), lambda i: (0, 0)),
            pl.BlockSpec((1, h), lambda i: (0, 0)),
            pl.BlockSpec((1, h), lambda i: (0, 0)),
        ],
        out_specs=pl.BlockSpec((bn, l, h), lambda i: (i, 0, 0)),
        out_shape=jax.ShapeDtypeStruct((n, l, h), jnp.float32),
        compiler_params=pltpu.CompilerParams(
            dimension_semantics=("arbitrary",),
        ),
    )(tok_c, dec_c, chg_c, table, pos_tile, gamma, beta)
    return out


# speedup vs baseline: 4.3879x; 1.0011x over previous
import functools

import jax
import jax.numpy as jnp
from jax.experimental import pallas as pl
from jax.experimental.pallas import tpu as pltpu

N, L, H = 1024, 50, 768
LN_EPS = 1e-12
BT = 1600
K = 128
HA = H + 128


def _body(tok_ref, dec_ref, chg_ref, lp_ref, table_ref, gamma_ref, beta_ref,
          out_ref):
    tok = tok_ref[...]
    dec = dec_ref[...]
    chg = chg_ref[...]
    lp = lp_ref[...]
    dec = dec - 5 * (dec == 5).astype(dec.dtype)
    col = jax.lax.broadcasted_iota(jnp.int32, (BT, K), 1)
    hot = ((col == tok) | (col == dec + 30) | (col == chg + 40)
           | (col == lp))
    hot = hot.astype(jnp.float32)
    xa = jnp.dot(hot, table_ref[...], preferred_element_type=jnp.float32)
    x = xa[:, :H]
    mean = xa[:, H:H + 1]
    ex2 = jnp.mean(x * x, axis=-1, keepdims=True)
    var = ex2 - mean * mean
    rstd = jax.lax.rsqrt(var + LN_EPS)
    y = (x * rstd - mean * rstd) * gamma_ref[...] + beta_ref[...]
    out_ref[...] = y.reshape(BT // L, L, H)


@jax.jit
def kernel(peptide_tokens, decoration, charge, a_emb, charge_emb, phos_emb,
           pos_emb, ln_gamma, ln_beta):
    n, l = peptide_tokens.shape
    h = a_emb.shape[1]
    nt = n * l
    tok_c = peptide_tokens.reshape(nt, 1).astype(jnp.int32)
    dec_c = decoration.reshape(nt, 1).astype(jnp.int32)
    chg_c = jnp.broadcast_to(charge[:, None], (n, l)).reshape(nt, 1)
    chg_c = chg_c.astype(jnp.int32)
    lp_c = jnp.tile(jnp.arange(l, dtype=jnp.int32) + 50, BT // l)
    lp_c = lp_c.reshape(BT, 1)
    table = jnp.concatenate(
        [a_emb, phos_emb, charge_emb, pos_emb[:l],
         jnp.zeros((K - 100, h), jnp.float32)], axis=0)
    msum = jnp.sum(table, axis=1, keepdims=True) / h
    table_aug = jnp.concatenate(
        [table, msum, jnp.zeros((K, HA - h - 1), jnp.float32)], axis=1)
    gamma = ln_gamma.reshape(1, h)
    beta = ln_beta.reshape(1, h)

    bn = BT // l
    grid = (nt // BT,)
    out = pl.pallas_call(
        _body,
        grid=grid,
        in_specs=[
            pl.BlockSpec((BT, 1), lambda i: (i, 0)),
            pl.BlockSpec((BT, 1), lambda i: (i, 0)),
            pl.BlockSpec((BT, 1), lambda i: (i, 0)),
            pl.BlockSpec((BT, 1), lambda i: (0, 0)),
            pl.BlockSpec((K, HA), lambda i: (0, 0)),
            pl.BlockSpec((1, h), lambda i: (0, 0)),
            pl.BlockSpec((1, h), lambda i: (0, 0)),
        ],
        out_specs=pl.BlockSpec((bn, l, h), lambda i: (i, 0, 0)),
        out_shape=jax.ShapeDtypeStruct((n, l, h), jnp.float32),
        compiler_params=pltpu.CompilerParams(
            dimension_semantics=("arbitrary",),
        ),
    )(tok_c, dec_c, chg_c, lp_c, table_aug, gamma, beta)
    return out


# speedup vs baseline: 5.5875x; 1.2734x over previous
import functools

import jax
import jax.numpy as jnp
from jax.experimental import pallas as pl
from jax.experimental.pallas import tpu as pltpu

N, L, H = 1024, 50, 768
LN_EPS = 1e-12
BN = 32
BT = BN * L
K = 128
HA = H + 128


def _body(code_ref, table_ref, gamma_ref, beta_ref, out_ref):
    code = code_ref[0]
    tok = code & 31
    dec = (code >> 5) & 31
    chg = code >> 10
    dec = dec - 5 * (dec == 5).astype(dec.dtype)
    lp = jax.lax.broadcasted_iota(jnp.int32, (1, BT), 1) % L + 50
    row = jax.lax.broadcasted_iota(jnp.int32, (K, BT), 0)
    hot = ((row == tok) | (row == dec + 30) | (row == chg + 40)
           | (row == lp))
    hot = hot.astype(jnp.float32)
    xa = jax.lax.dot_general(
        hot, table_ref[...], (((0,), (0,)), ((), ())),
        preferred_element_type=jnp.float32)
    x = xa[:, :H]
    mean = xa[:, H:H + 1]
    ex2 = jnp.mean(x * x, axis=-1, keepdims=True)
    var = ex2 - mean * mean
    rstd = jax.lax.rsqrt(var + LN_EPS)
    y = (x * rstd - mean * rstd) * gamma_ref[...] + beta_ref[...]
    out_ref[...] = y.reshape(BN, L, H)


@jax.jit
def kernel(peptide_tokens, decoration, charge, a_emb, charge_emb, phos_emb,
           pos_emb, ln_gamma, ln_beta):
    n, l = peptide_tokens.shape
    h = a_emb.shape[1]
    code = (peptide_tokens.astype(jnp.int32)
            | (decoration.astype(jnp.int32) << 5)
            | (charge.astype(jnp.int32)[:, None] << 10))
    code = code.reshape(n // BN, 1, BT)
    table = jnp.concatenate(
        [a_emb, phos_emb, charge_emb, pos_emb[:l],
         jnp.zeros((K - 100, h), jnp.float32)], axis=0)
    msum = jnp.sum(table, axis=1, keepdims=True) / h
    table_aug = jnp.concatenate(
        [table, msum, jnp.zeros((K, HA - h - 1), jnp.float32)], axis=1)
    gamma = ln_gamma.reshape(1, h)
    beta = ln_beta.reshape(1, h)

    grid = (n // BN,)
    out = pl.pallas_call(
        _body,
        grid=grid,
        in_specs=[
            pl.BlockSpec((1, 1, BT), lambda i: (i, 0, 0)),
            pl.BlockSpec((K, HA), lambda i: (0, 0)),
            pl.BlockSpec((1, h), lambda i: (0, 0)),
            pl.BlockSpec((1, h), lambda i: (0, 0)),
        ],
        out_specs=pl.BlockSpec((BN, l, h), lambda i: (i, 0, 0)),
        out_shape=jax.ShapeDtypeStruct((n, l, h), jnp.float32),
        compiler_params=pltpu.CompilerParams(
            dimension_semantics=("arbitrary",),
        ),
    )(code, table_aug, gamma, beta)
    return out


# speedup vs baseline: 11.7562x; 2.1040x over previous
import functools

import jax
import jax.numpy as jnp
from jax.experimental import pallas as pl
from jax.experimental.pallas import tpu as pltpu

N, L, H = 1024, 50, 768
LN_EPS = 1e-12
BL = 2
BT = BL * N
K = 128


def _body(code_ref, table_ref, msum_ref, gamma_ref, beta_ref, out_ref):
    code = code_ref[0]
    tok = code & 31
    dec = (code >> 5) & 31
    chg = code >> 10
    dec = dec - 5 * (dec == 5).astype(dec.dtype)
    lp = (jax.lax.broadcasted_iota(jnp.int32, (1, BT), 1) // N
          + BL * pl.program_id(0) + 50)
    row = jax.lax.broadcasted_iota(jnp.int32, (K, BT), 0)
    hot = ((row == tok) | (row == dec + 30) | (row == chg + 40)
           | (row == lp))
    hot = hot.astype(jnp.float32)
    x = jax.lax.dot_general(
        hot, table_ref[...], (((0,), (0,)), ((), ())),
        preferred_element_type=jnp.float32)
    mm = jax.lax.dot_general(
        hot, msum_ref[...], (((0,), (0,)), ((), ())),
        preferred_element_type=jnp.float32)
    mean = mm[:, :1]
    ex2 = jnp.mean(x * x, axis=-1, keepdims=True)
    var = ex2 - mean * mean
    rstd = jax.lax.rsqrt(var + LN_EPS)
    y = (x * rstd - mean * rstd) * gamma_ref[...] + beta_ref[...]
    out_ref[...] = y.reshape(BL, N, H)


@jax.jit
def kernel(peptide_tokens, decoration, charge, a_emb, charge_emb, phos_emb,
           pos_emb, ln_gamma, ln_beta):
    n, l = peptide_tokens.shape
    h = a_emb.shape[1]
    code = (peptide_tokens.T.astype(jnp.int32)
            | (decoration.T.astype(jnp.int32) << 5)
            | (charge.astype(jnp.int32)[None, :] << 10))
    code = code.reshape(l // BL, 1, BT)
    table = jnp.concatenate(
        [a_emb, phos_emb, charge_emb, pos_emb[:l],
         jnp.zeros((K - 100, h), jnp.float32)], axis=0)
    msum = jnp.concatenate(
        [jnp.sum(table, axis=1, keepdims=True) / h,
         jnp.zeros((K, 127), jnp.float32)], axis=1)
    gamma = ln_gamma.reshape(1, h)
    beta = ln_beta.reshape(1, h)

    grid = (l // BL,)
    out = pl.pallas_call(
        _body,
        grid=grid,
        in_specs=[
            pl.BlockSpec((1, 1, BT), lambda i: (i, 0, 0)),
            pl.BlockSpec((K, h), lambda i: (0, 0)),
            pl.BlockSpec((K, K), lambda i: (0, 0)),
            pl.BlockSpec((1, h), lambda i: (0, 0)),
            pl.BlockSpec((1, h), lambda i: (0, 0)),
        ],
        out_specs=pl.BlockSpec((BL, n, h), lambda i: (i, 0, 0)),
        out_shape=jax.ShapeDtypeStruct((l, n, h), jnp.float32),
        compiler_params=pltpu.CompilerParams(
            dimension_semantics=("arbitrary",),
        ),
    )(code, table, msum, gamma, beta)
    return out.transpose(1, 0, 2)
